# Initial kernel scaffold; baseline (speedup 1.0000x reference)
import jax
import jax.numpy as jnp
from jax.experimental import pallas as pl


def kernel(x, conv_w, W1, b1, W2, b2, out_w):
    raise NotImplementedError("write your pallas kernel here")


# speedup vs baseline: 2.0536x; 2.0536x over previous
import functools

import jax
import jax.numpy as jnp
from jax import lax
from jax.experimental import pallas as pl
from jax.experimental.pallas import tpu as pltpu
from jax.experimental.pallas import tpu_sc as plsc

_N, _C, _RES, _F = 32, 192, 32, 4
_LH = _RES // _F
_HW = _LH * _LH
_NB = _F * _F
_K = 6
_R = _NB * _N
_INDIM = _K * _C + _HW
_HID = _INDIM
_OC = _C // 10 + 1
_OUTD = _HW * _OC

_SC_NC, _SC_NS = 2, 16
_NW = _SC_NC * _SC_NS
_NIDX = _R * _K
_BPW = _NIDX // _NW
_CP = 256


def _scores_body(xb_ref, b_ref, t_ref):
    a = xb_ref[0].astype(jnp.bfloat16)
    b = b_ref[0].astype(jnp.bfloat16)
    t_ref[0] = jnp.dot(a, b, preferred_element_type=jnp.float32)


def _run_scores(xb3, bmat):
    return pl.pallas_call(
        _scores_body,
        grid=(_NB,),
        in_specs=[
            pl.BlockSpec((1, _N * _HW, _C), lambda i: (i, 0, 0)),
            pl.BlockSpec((1, _C, 8), lambda i: (i, 0, 0)),
        ],
        out_specs=pl.BlockSpec((1, _N * _HW, 8), lambda i: (i, 0, 0)),
        out_shape=jax.ShapeDtypeStruct((_NB, _N * _HW, 8), jnp.float32),
    )(xb3, bmat)


def _select_body(t_ref, mask_ref, gidx_ref):
    bi = pl.program_id(0)
    t = t_ref[0]
    e = jnp.exp(t)
    normed = e / jnp.sum(e, axis=1, keepdims=True)
    pos = lax.broadcasted_iota(jnp.int32, (_N, _HW), 1)
    work = normed
    mask = jnp.zeros((_N, _HW), jnp.float32)
    for _ in range(_K):
        m = jnp.max(work, axis=1, keepdims=True)
        is_max = work == m
        cand = jnp.where(is_max, pos, _HW)
        sel = jnp.min(cand, axis=1, keepdims=True)
        one = pos == sel
        mask = mask + one.astype(jnp.float32)
        work = jnp.where(one, -jnp.inf, work)
    mask_ref[0] = mask
    candp = jnp.where(mask > 0.5, pos, _HW)
    rowbase = (bi * _N + lax.broadcasted_iota(jnp.int32, (_N, 1), 0)) * _HW
    cols = []
    for _ in range(_K):
        mn = jnp.min(candp, axis=1, keepdims=True)
        cols.append(rowbase + mn)
        candp = jnp.where(candp == mn, _HW, candp)
    gidx_ref[0] = jnp.concatenate(cols, axis=1)


def _run_select(t):
    mask, gidx = pl.pallas_call(
        _select_body,
        grid=(_NB,),
        in_specs=[
            pl.BlockSpec((1, _N, _HW), lambda i: (i, 0, 0)),
        ],
        out_specs=[
            pl.BlockSpec((1, _N, _HW), lambda i: (i, 0, 0)),
            pl.BlockSpec((1, _N, _K), lambda i: (i, 0, 0)),
        ],
        out_shape=[
            jax.ShapeDtypeStruct((_NB, _N, _HW), jnp.float32),
            jax.ShapeDtypeStruct((_NB, _N, _K), jnp.int32),
        ],
    )(t)
    return mask, gidx


def _make_sc_gather():
    mesh = plsc.VectorSubcoreMesh(core_axis_name="c", subcore_axis_name="s")

    @functools.partial(
        pl.kernel,
        mesh=mesh,
        out_type=jax.ShapeDtypeStruct((_NIDX, _CP), jnp.float32),
        scratch_types=[
            pltpu.VMEM((_BPW,), jnp.int32),
            pltpu.VMEM((_BPW, _CP), jnp.float32),
            pltpu.SemaphoreType.DMA,
        ],
    )
    def gather_k(table_hbm, idx_hbm, out_hbm, idx_v, rows_v, sem):
        wid = lax.axis_index("s") * _SC_NC + lax.axis_index("c")
        base = wid * _BPW
        pltpu.sync_copy(idx_hbm.at[pl.ds(base, _BPW)], idx_v)
        pltpu.async_copy(table_hbm.at[idx_v], rows_v, sem).wait()
        pltpu.sync_copy(rows_v, out_hbm.at[pl.ds(base, _BPW)])

    return gather_k


_sc_gather_impl = None


def _sc_gather(table, idx):
    global _sc_gather_impl
    if _sc_gather_impl is None:
        _sc_gather_impl = _make_sc_gather()
    return _sc_gather_impl(table, idx)


def _mlp_body(cin_ref, w1t_ref, b1_ref, w2t_ref, b2_ref, out_ref):
    hid = jnp.dot(cin_ref[...], w1t_ref[...], preferred_element_type=jnp.float32)
    hid = jnp.maximum(hid + b1_ref[...], 0.0)
    out_ref[...] = (
        jnp.dot(hid, w2t_ref[...], preferred_element_type=jnp.float32)
        + b2_ref[...]
    )


def _run_mlp(cin, W1, b1, W2, b2):
    return pl.pallas_call(
        _mlp_body,
        out_shape=jax.ShapeDtypeStruct((_R, _OUTD), jnp.float32),
    )(cin, W1.T, b1.reshape(1, _HID), W2.T, b2.reshape(1, _OUTD))


def _outconv_body(full_ref, ow_ref, out_ref):
    out_ref[0] = jnp.dot(
        ow_ref[...], full_ref[0], preferred_element_type=jnp.float32
    )


def _run_outconv(full, out_w):
    return pl.pallas_call(
        _outconv_body,
        grid=(_N,),
        in_specs=[
            pl.BlockSpec((1, _OC, _RES * _RES), lambda i: (i, 0, 0)),
            pl.BlockSpec((_C, _OC), lambda i: (0, 0)),
        ],
        out_specs=pl.BlockSpec((1, _C, _RES * _RES), lambda i: (i, 0, 0)),
        out_shape=jax.ShapeDtypeStruct((_N, _C, _RES * _RES), jnp.float32),
    )(full, out_w)


def kernel(x, conv_w, W1, b1, W2, b2, out_w):
    n, c, h, w = x.shape
    xb_flat = (
        x.reshape(n, c, _F, _LH, _F, _LH)
        .transpose(2, 4, 0, 3, 5, 1)
        .reshape(_NB, _N, _HW, _C)
    )
    bmat = jnp.zeros((_NB, _C, 8), jnp.float32).at[:, :, 0].set(conv_w)
    t8 = _run_scores(xb_flat.reshape(_NB, _N * _HW, _C), bmat)
    t = t8[:, :, 0].reshape(_NB, _N, _HW)
    mask, gidx = _run_select(t)

    table = jnp.pad(
        xb_flat.reshape(_R * _HW, _C), ((0, 0), (0, _CP - _C))
    )
    gathered = _sc_gather(table, gidx.reshape(_NIDX))

    cin = jnp.concatenate(
        [gathered[:, :_C].reshape(_R, _K * _C), mask.reshape(_R, _HW)], axis=1
    )
    rec = _run_mlp(cin, W1, b1, W2, b2)

    full = (
        rec.reshape(_F, _F, _N, _OC, _LH, _LH)
        .transpose(2, 3, 0, 4, 1, 5)
        .reshape(_N, _OC, _RES * _RES)
    )
    out = _run_outconv(full, out_w)
    return out.reshape(_N, _C, _RES, _RES)
